# baseline (speedup 1.0000x reference)
import functools

import jax
import jax.numpy as jnp
from jax import lax
from jax.experimental import pallas as pl
from jax.experimental.pallas import tpu as pltpu
from jax.experimental.pallas import tpu_sc as plsc

SCALE = 100.0
G = 64
ND = 16
S = 16
F = S * ND
NB = 1000
EB = 16000
NWK = 32


def _node_body(x_ref, nw_ref, b_ref, v_ref, slin_ref, rs_ref,
               nh_ref, acc_ref, st_ref):
    step = pl.program_id(0)
    xw = x_ref[...] * nw_ref[...]
    nh = jnp.dot(xw, v_ref[...], preferred_element_type=jnp.float32)
    nh_ref[...] = nh
    zh = jnp.dot(nh, rs_ref[...], preferred_element_type=jnp.float32)
    sig = 0.5 + 0.5 * jnp.tanh(slin_ref[...] - zh)
    sigb = sig.astype(jnp.bfloat16)
    b = b_ref[0]
    gio = lax.broadcasted_iota(jnp.int32, (G, NB), 0)
    onehot = jnp.where(b == gio, 1.0, 0.0).astype(jnp.bfloat16)

    @pl.when(step == 0)
    def _():
        acc_ref[...] = jnp.zeros_like(acc_ref)
        st_ref[...] = jnp.zeros_like(st_ref)

    acc_ref[...] += jnp.dot(onehot, sigb, preferred_element_type=jnp.float32)
    st_ref[...] += jnp.sum((b < gio).astype(jnp.int32), axis=1,
                           keepdims=True)


def _node_pass(x, nw2, bidx_lanes, v, slin, rs, *, interpret=False):
    n, dim = x.shape
    nsteps = n // NB
    return pl.pallas_call(
        _node_body,
        grid=(nsteps,),
        in_specs=[
            pl.BlockSpec((NB, dim), lambda i: (i, 0)),
            pl.BlockSpec((NB, 1), lambda i: (i, 0)),
            pl.BlockSpec((1, 1, NB), lambda i: (i, 0, 0)),
            pl.BlockSpec((dim, ND), lambda i: (0, 0)),
            pl.BlockSpec((1, F), lambda i: (0, 0)),
            pl.BlockSpec((ND, F), lambda i: (0, 0)),
        ],
        out_specs=[
            pl.BlockSpec((NB, ND), lambda i: (i, 0)),
            pl.BlockSpec((G, F), lambda i: (0, 0)),
            pl.BlockSpec((G, 1), lambda i: (0, 0)),
        ],
        out_shape=[
            jax.ShapeDtypeStruct((n, ND), jnp.float32),
            jax.ShapeDtypeStruct((G, F), jnp.float32),
            jax.ShapeDtypeStruct((G, 1), jnp.int32),
        ],
        compiler_params=pltpu.CompilerParams(
            dimension_semantics=("arbitrary",)),
        interpret=interpret,
    )(x, nw2, bidx_lanes, v, slin, rs)


def _sc_gather(nh, edge_index):
    e = edge_index.shape[1]
    bpw = e // NWK
    mesh = plsc.VectorSubcoreMesh(core_axis_name="c", subcore_axis_name="s")

    @functools.partial(
        pl.kernel,
        out_type=(jax.ShapeDtypeStruct((e, ND), jnp.float32),
                  jax.ShapeDtypeStruct((e, ND), jnp.float32)),
        mesh=mesh,
        scratch_types=[
            pltpu.VMEM((bpw,), jnp.int32),
            pltpu.VMEM((bpw, ND), jnp.float32),
            pltpu.SemaphoreType.DMA,
        ],
        compiler_params=pltpu.CompilerParams(use_tc_tiling_on_sc=False),
    )
    def k(nh_hbm, ei_hbm, uout, vout, idx_v, rows_v, sem):
        wid = lax.axis_index("s") * 2 + lax.axis_index("c")
        base = wid * bpw
        pltpu.sync_copy(ei_hbm.at[0, pl.ds(base, bpw)], idx_v)
        pltpu.async_copy(nh_hbm.at[idx_v], rows_v, sem).wait()
        pltpu.sync_copy(rows_v, uout.at[pl.ds(base, bpw)])
        pltpu.sync_copy(ei_hbm.at[1, pl.ds(base, bpw)], idx_v)
        pltpu.async_copy(nh_hbm.at[idx_v], rows_v, sem).wait()
        pltpu.sync_copy(rows_v, vout.at[pl.ds(base, bpw)])

    return k(nh, edge_index)


def _edge_body(u_ref, v_ref, ul_ref, st_ref, accp_ref,
               slin_ref, rs_ref, f1w_ref, f1b_ref, f2w_ref, f2b_ref,
               logits_ref, flat_ref, acc_scr):
    step = pl.program_id(0)

    @pl.when(step == 0)
    def _():
        acc_scr[...] = jnp.zeros_like(acc_scr)

    m = jnp.maximum(u_ref[...], v_ref[...])
    ewu = ul_ref[0]
    io8 = lax.broadcasted_iota(jnp.int32, (8, 8), 0)
    i8 = jnp.where(io8 == io8.T, 1.0, 0.0).astype(jnp.float32)
    u8 = lax.dot_general(i8, ewu[:, 8:16], (((1,), (1,)), ((), ())),
                         precision=lax.Precision.HIGHEST,
                         preferred_element_type=jnp.float32)
    st = st_ref[...].astype(jnp.float32)
    acc = jnp.zeros_like(acc_scr)
    for k in range(8):
        mk = m[:, ND * k:ND * (k + 1)]
        zh = jnp.dot(mk, rs_ref[...],
                     preferred_element_type=jnp.float32) * ewu[:, k:k + 1]
        sig = 0.5 + 0.5 * jnp.tanh(slin_ref[...] - zh)
        sigb = sig.astype(jnp.bfloat16)
        uk = u8[k:k + 1, :]
        ge = jnp.where(uk >= st, 1.0, 0.0).astype(jnp.bfloat16)
        acc = acc + jnp.dot(ge, sigb, preferred_element_type=jnp.float32)
    accd = acc - jnp.concatenate(
        [acc[1:, :], jnp.zeros((1, F), jnp.float32)], axis=0)
    acc_scr[...] += accd

    @pl.when(step == pl.num_programs(0) - 1)
    def _():
        flat = accp_ref[...] - acc_scr[...]
        flat_ref[...] = flat
        h = jnp.maximum(
            lax.dot_general(flat, f1w_ref[...], (((1,), (1,)), ((), ())),
                            preferred_element_type=jnp.float32)
            + f1b_ref[...], 0.0)
        logits_ref[...] = (
            lax.dot_general(h, f2w_ref[...], (((1,), (1,)), ((), ())),
                            preferred_element_type=jnp.float32)
            + f2b_ref[...])


def _edge_pass(u_pack, v_pack, ewu, starts, acc_pts,
               slin, rs, f1w, f1b2, f2w, f2b2, *, interpret=False):
    e = u_pack.shape[0] * 128 // ND
    nsteps = e // EB
    hid = f1w.shape[0]
    ncls = f2w.shape[0]
    return pl.pallas_call(
        _edge_body,
        grid=(nsteps,),
        in_specs=[
            pl.BlockSpec((EB * ND // 128, 128), lambda i: (i, 0)),
            pl.BlockSpec((EB * ND // 128, 128), lambda i: (i, 0)),
            pl.BlockSpec((1, EB // 8, 16), lambda i: (i, 0, 0)),
            pl.BlockSpec((G, 1), lambda i: (0, 0)),
            pl.BlockSpec((G, F), lambda i: (0, 0)),
            pl.BlockSpec((1, F), lambda i: (0, 0)),
            pl.BlockSpec((ND, F), lambda i: (0, 0)),
            pl.BlockSpec((hid, F), lambda i: (0, 0)),
            pl.BlockSpec((1, hid), lambda i: (0, 0)),
            pl.BlockSpec((ncls, hid), lambda i: (0, 0)),
            pl.BlockSpec((1, ncls), lambda i: (0, 0)),
        ],
        out_specs=[
            pl.BlockSpec((G, ncls), lambda i: (0, 0)),
            pl.BlockSpec((G, F), lambda i: (0, 0)),
        ],
        out_shape=[
            jax.ShapeDtypeStruct((G, ncls), jnp.float32),
            jax.ShapeDtypeStruct((G, F), jnp.float32),
        ],
        scratch_shapes=[pltpu.VMEM((G, F), jnp.float32)],
        compiler_params=pltpu.CompilerParams(
            dimension_semantics=("arbitrary",)),
        interpret=interpret,
    )(u_pack, v_pack, ewu, starts, acc_pts,
      slin, rs, f1w, f1b2, f2w, f2b2)


def kernel(x, node_weights, edge_index, edge_weights, batch_idx, v, lin,
           fc1_w, fc1_b, fc2_w, fc2_b):
    n = x.shape[0]
    e = edge_index.shape[1]
    rs = (0.5 * SCALE) * jnp.tile(jnp.eye(ND, dtype=jnp.float32), (1, S))
    slin = ((0.5 * SCALE) * jnp.repeat(lin, ND))[None, :]
    nw2 = node_weights[:, None]
    bidx_lanes = batch_idx.reshape(n // NB, 1, NB)
    ewu = jnp.concatenate(
        [edge_weights.reshape(e // EB, EB // 8, 8),
         edge_index[0].astype(jnp.float32).reshape(e // EB, EB // 8, 8)],
        axis=2)

    nh, acc_pts, starts = _node_pass(x, nw2, bidx_lanes, v, slin, rs)
    u_rows, v_rows = _sc_gather(nh, edge_index)
    u_pack = u_rows.reshape(e * ND // 128, 128)
    v_pack = v_rows.reshape(e * ND // 128, 128)
    logits, flat = _edge_pass(
        u_pack, v_pack, ewu, starts, acc_pts,
        slin, rs, fc1_w, fc1_b[None, :], fc2_w, fc2_b[None, :])
    return (logits, flat)


# speedup vs baseline: 1.1082x; 1.1082x over previous
import functools

import jax
import jax.numpy as jnp
from jax import lax
from jax.experimental import pallas as pl
from jax.experimental.pallas import tpu as pltpu
from jax.experimental.pallas import tpu_sc as plsc

SCALE = 100.0
G = 64
ND = 16
S = 16
F = S * ND
NB = 1000
EB = 16000
NWK = 32


def _node_body(x_ref, nw_ref, b_ref, v_ref, slin_ref, rs_ref,
               nh_ref, acc_ref, st_ref):
    step = pl.program_id(0)
    xw = x_ref[...] * nw_ref[...]
    nh = jnp.dot(xw, v_ref[...], preferred_element_type=jnp.float32)
    nh_ref[...] = nh
    zh = jnp.dot(nh, rs_ref[...], preferred_element_type=jnp.float32)
    sig = 0.5 + 0.5 * jnp.tanh(slin_ref[...] - zh)
    sigb = sig.astype(jnp.bfloat16)
    b = b_ref[0]
    gio = lax.broadcasted_iota(jnp.int32, (G, NB), 0)
    onehot = jnp.where(b == gio, 1.0, 0.0).astype(jnp.bfloat16)

    @pl.when(step == 0)
    def _():
        acc_ref[...] = jnp.zeros_like(acc_ref)
        st_ref[...] = jnp.zeros_like(st_ref)

    acc_ref[...] += jnp.dot(onehot, sigb, preferred_element_type=jnp.float32)
    st_ref[...] += jnp.sum((b < gio).astype(jnp.int32), axis=1,
                           keepdims=True)


def _node_pass(x, nw2, bidx_lanes, v, slin, rs, *, interpret=False):
    n, dim = x.shape
    nsteps = n // NB
    return pl.pallas_call(
        _node_body,
        grid=(nsteps,),
        in_specs=[
            pl.BlockSpec((NB, dim), lambda i: (i, 0)),
            pl.BlockSpec((NB, 1), lambda i: (i, 0)),
            pl.BlockSpec((1, 1, NB), lambda i: (i, 0, 0)),
            pl.BlockSpec((dim, ND), lambda i: (0, 0)),
            pl.BlockSpec((1, F), lambda i: (0, 0)),
            pl.BlockSpec((ND, F), lambda i: (0, 0)),
        ],
        out_specs=[
            pl.BlockSpec((NB, ND), lambda i: (i, 0)),
            pl.BlockSpec((G, F), lambda i: (0, 0)),
            pl.BlockSpec((G, 1), lambda i: (0, 0)),
        ],
        out_shape=[
            jax.ShapeDtypeStruct((n, ND), jnp.float32),
            jax.ShapeDtypeStruct((G, F), jnp.float32),
            jax.ShapeDtypeStruct((G, 1), jnp.int32),
        ],
        compiler_params=pltpu.CompilerParams(
            dimension_semantics=("arbitrary",)),
        interpret=interpret,
    )(x, nw2, bidx_lanes, v, slin, rs)


def _sc_gather(nh, edge_index):
    e = edge_index.shape[1]
    bpw = e // NWK
    mesh = plsc.VectorSubcoreMesh(core_axis_name="c", subcore_axis_name="s")

    @functools.partial(
        pl.kernel,
        out_type=(jax.ShapeDtypeStruct((e, ND), jnp.float32),
                  jax.ShapeDtypeStruct((e, ND), jnp.float32)),
        mesh=mesh,
        scratch_types=[
            pltpu.VMEM((bpw,), jnp.int32),
            pltpu.VMEM((bpw, ND), jnp.float32),
            pltpu.SemaphoreType.DMA,
        ],
        compiler_params=pltpu.CompilerParams(use_tc_tiling_on_sc=False),
    )
    def k(nh_hbm, ei_hbm, uout, vout, idx_v, rows_v, sem):
        wid = lax.axis_index("s") * 2 + lax.axis_index("c")
        base = wid * bpw
        pltpu.sync_copy(ei_hbm.at[0, pl.ds(base, bpw)], idx_v)
        pltpu.async_copy(nh_hbm.at[idx_v], rows_v, sem).wait()
        pltpu.sync_copy(rows_v, uout.at[pl.ds(base, bpw)])
        pltpu.sync_copy(ei_hbm.at[1, pl.ds(base, bpw)], idx_v)
        pltpu.async_copy(nh_hbm.at[idx_v], rows_v, sem).wait()
        pltpu.sync_copy(rows_v, vout.at[pl.ds(base, bpw)])

    return k(nh, edge_index)


def _edge_body(u_ref, v_ref, ew_ref, ul_ref, st_ref, accp_ref,
               slin_ref, rs_ref, f1w_ref, f1b_ref, f2w_ref, f2b_ref,
               logits_ref, flat_ref, acc_scr):
    step = pl.program_id(0)

    @pl.when(step == 0)
    def _():
        acc_scr[...] = jnp.zeros_like(acc_scr)

    m = jnp.maximum(u_ref[...], v_ref[...])
    ew8 = ew_ref[0]
    u8 = ul_ref[0]
    st = st_ref[...]
    acc = jnp.zeros_like(acc_scr)
    for k in range(8):
        mk = m[:, ND * k:ND * (k + 1)]
        zh = jnp.dot(mk, rs_ref[...],
                     preferred_element_type=jnp.float32) * ew8[:, k:k + 1]
        sig = 0.5 + 0.5 * jnp.tanh(slin_ref[...] - zh)
        sigb = sig.astype(jnp.bfloat16)
        uk = u8[k:k + 1, :]
        ge = jnp.where(uk >= st, 1.0, 0.0).astype(jnp.bfloat16)
        acc = acc + jnp.dot(ge, sigb, preferred_element_type=jnp.float32)
    accd = acc - jnp.concatenate(
        [acc[1:, :], jnp.zeros((1, F), jnp.float32)], axis=0)
    acc_scr[...] += accd

    @pl.when(step == pl.num_programs(0) - 1)
    def _():
        flat = accp_ref[...] - acc_scr[...]
        flat_ref[...] = flat
        h = jnp.maximum(
            lax.dot_general(flat, f1w_ref[...], (((1,), (1,)), ((), ())),
                            preferred_element_type=jnp.float32)
            + f1b_ref[...], 0.0)
        logits_ref[...] = (
            lax.dot_general(h, f2w_ref[...], (((1,), (1,)), ((), ())),
                            preferred_element_type=jnp.float32)
            + f2b_ref[...])


def _edge_pass(u_pack, v_pack, ew2, u_lanes, starts, acc_pts,
               slin, rs, f1w, f1b2, f2w, f2b2, *, interpret=False):
    e = u_pack.shape[0] * 128 // ND
    nsteps = e // EB
    hid = f1w.shape[0]
    ncls = f2w.shape[0]
    return pl.pallas_call(
        _edge_body,
        grid=(nsteps,),
        in_specs=[
            pl.BlockSpec((EB * ND // 128, 128), lambda i: (i, 0)),
            pl.BlockSpec((EB * ND // 128, 128), lambda i: (i, 0)),
            pl.BlockSpec((1, EB // 8, 8), lambda i: (i, 0, 0)),
            pl.BlockSpec((1, 8, EB // 8), lambda i: (i, 0, 0)),
            pl.BlockSpec((G, 1), lambda i: (0, 0)),
            pl.BlockSpec((G, F), lambda i: (0, 0)),
            pl.BlockSpec((1, F), lambda i: (0, 0)),
            pl.BlockSpec((ND, F), lambda i: (0, 0)),
            pl.BlockSpec((hid, F), lambda i: (0, 0)),
            pl.BlockSpec((1, hid), lambda i: (0, 0)),
            pl.BlockSpec((ncls, hid), lambda i: (0, 0)),
            pl.BlockSpec((1, ncls), lambda i: (0, 0)),
        ],
        out_specs=[
            pl.BlockSpec((G, ncls), lambda i: (0, 0)),
            pl.BlockSpec((G, F), lambda i: (0, 0)),
        ],
        out_shape=[
            jax.ShapeDtypeStruct((G, ncls), jnp.float32),
            jax.ShapeDtypeStruct((G, F), jnp.float32),
        ],
        scratch_shapes=[pltpu.VMEM((G, F), jnp.float32)],
        compiler_params=pltpu.CompilerParams(
            dimension_semantics=("arbitrary",)),
        interpret=interpret,
    )(u_pack, v_pack, ew2, u_lanes, starts, acc_pts,
      slin, rs, f1w, f1b2, f2w, f2b2)


def kernel(x, node_weights, edge_index, edge_weights, batch_idx, v, lin,
           fc1_w, fc1_b, fc2_w, fc2_b):
    n = x.shape[0]
    e = edge_index.shape[1]
    rs = (0.5 * SCALE) * jnp.tile(jnp.eye(ND, dtype=jnp.float32), (1, S))
    slin = ((0.5 * SCALE) * jnp.repeat(lin, ND))[None, :]
    nw2 = node_weights[:, None]
    bidx_lanes = batch_idx.reshape(n // NB, 1, NB)
    ew2 = edge_weights.reshape(e // EB, EB // 8, 8)
    u_lanes = edge_index[0].reshape(e // EB, EB // 8, 8).transpose(0, 2, 1)

    nh, acc_pts, starts = _node_pass(x, nw2, bidx_lanes, v, slin, rs)
    u_rows, v_rows = _sc_gather(nh, edge_index)
    u_pack = u_rows.reshape(e * ND // 128, 128)
    v_pack = v_rows.reshape(e * ND // 128, 128)
    logits, flat = _edge_pass(
        u_pack, v_pack, ew2, u_lanes, starts, acc_pts,
        slin, rs, fc1_w, fc1_b[None, :], fc2_w, fc2_b[None, :])
    return (logits, flat)
